# baseline (speedup 1.0000x reference)
import functools

import jax
import jax.numpy as jnp
from jax import lax
from jax.experimental import pallas as pl
from jax.experimental.pallas import tpu as pltpu
from jax.experimental.pallas import tpu_sc as plsc

VOCAB = 100000
EMB = 64
BATCH = 16384

NC = 2
NS = 16
NW = NC * NS
BPW = BATCH // NW
CH = 128
NCH = BPW // CH


KG = 16
NIT = BPW // KG


def _sc_gather_body(user_h, item_h, su_h, tu_h, si_h, ti_h,
                    osu, otu, osi, oti, idx_u, idx_i, sem):
    c = lax.axis_index("c")
    s = lax.axis_index("s")
    wid = s * NC + c
    base = wid * BPW
    pltpu.sync_copy(user_h.at[pl.ds(base, BPW)], idx_u)
    pltpu.sync_copy(item_h.at[pl.ds(base, BPW)], idx_i)

    for idxbuf, tbl_a, tbl_b, out_a, out_b in (
            (idx_u, su_h, tu_h, osu, otu),
            (idx_i, si_h, ti_h, osi, oti)):
        def body(j, carry):
            o = j * KG
            v = idxbuf[pl.ds(o, KG)]
            for i in range(KG):
                r = v[i]
                dst = pl.ds(base + o + i, 1)
                pltpu.async_copy(tbl_a.at[pl.ds(r, 1)], out_a.at[dst], sem)
                pltpu.async_copy(tbl_b.at[pl.ds(r, 1)], out_b.at[dst], sem)

            @pl.when(j > 0)
            def _():
                pltpu.make_async_copy(out_a.at[pl.ds(base, 2 * KG)],
                                      out_b.at[pl.ds(base, 2 * KG)],
                                      sem).wait()

            return carry

        lax.fori_loop(0, NIT, body, 0)
        pltpu.make_async_copy(out_a.at[pl.ds(base, 2 * KG)],
                              out_b.at[pl.ds(base, 2 * KG)], sem).wait()


@functools.lru_cache(maxsize=1)
def _sc_gather():
    return pl.kernel(
        _sc_gather_body,
        out_type=tuple(jax.ShapeDtypeStruct((BATCH, EMB), jnp.float32)
                       for _ in range(4)),
        mesh=plsc.VectorSubcoreMesh(core_axis_name="c", subcore_axis_name="s",
                                    num_cores=NC, num_subcores=NS),
        scratch_types=[
            pltpu.VMEM((BPW,), jnp.int32),
            pltpu.VMEM((BPW,), jnp.int32),
            pltpu.SemaphoreType.DMA,
        ],
    )


BLK = 2048


def _mlp_body(g_su, g_tu, g_si, g_ti, W1, b1, W2, b2, Wp, bp, out):
    ue = jnp.maximum(g_su[...], g_tu[...])
    ie = jnp.maximum(g_si[...], g_ti[...])
    h = jnp.concatenate((ue, ie), axis=1)
    h = jnp.dot(h, W1[...], preferred_element_type=jnp.float32) + b1[...]
    h = jnp.maximum(h, 0.0)
    h = jnp.dot(h, W2[...], preferred_element_type=jnp.float32) + b2[...]
    h = jnp.maximum(h, 0.0)
    o = jnp.dot(h, Wp[...], preferred_element_type=jnp.float32) + bp[...]
    out[...] = jax.nn.sigmoid(o)


def _row_spec():
    return pl.BlockSpec((BLK, EMB), lambda i: (i, 0))


def _full_spec(shape):
    return pl.BlockSpec(shape, lambda i: tuple(0 for _ in shape))


_mlp = pl.pallas_call(
    _mlp_body,
    grid=(BATCH // BLK,),
    in_specs=[
        _row_spec(), _row_spec(), _row_spec(), _row_spec(),
        _full_spec((2 * EMB, 128)), _full_spec((1, 128)),
        _full_spec((128, 64)), _full_spec((1, 64)),
        _full_spec((64, 1)), _full_spec((1, 1)),
    ],
    out_specs=pl.BlockSpec((BLK, 1), lambda i: (i, 0)),
    out_shape=jax.ShapeDtypeStruct((BATCH, 1), jnp.float32),
)


@jax.jit
def kernel(x, su_emb, tu_emb, si_emb, ti_emb, W1, b1, W2, b2, Wp, bp):
    x = x.astype(jnp.int32)
    user = x[:, 0]
    item = x[:, 1]
    g_su, g_tu, g_si, g_ti = _sc_gather()(user, item, su_emb, tu_emb,
                                          si_emb, ti_emb)
    out = _mlp(g_su, g_tu, g_si, g_ti,
               W1, b1.reshape(1, -1), W2, b2.reshape(1, -1),
               Wp, bp.reshape(1, 1))
    return out[:, 0]


# speedup vs baseline: 3.4800x; 3.4800x over previous
import functools

import jax
import jax.numpy as jnp
from jax import lax
from jax.experimental import pallas as pl
from jax.experimental.pallas import tpu as pltpu
from jax.experimental.pallas import tpu_sc as plsc

VOCAB = 100000
EMB = 64
BATCH = 16384

NC = 2
NS = 16
DPT = EMB // NS
PASS = 20000
NPASS = VOCAB // PASS
L = 16
VECS = BATCH // L


def _sc_gather_body(user_h, item_h, suT, tuT, siT, tiT, uT, iT,
                    idx_v, acc, ca0, cb0, ca1, cb1, s0, s1):
    c = lax.axis_index("c")
    s = lax.axis_index("s")

    cols = ((ca0, cb0, s0), (ca1, cb1, s1))

    def run_pair(idx_src, tbl_a, tbl_b, out):
        pltpu.sync_copy(idx_src, idx_v)
        for k in range(DPT):
            d = s * DPT + k

            cps = [None, None]

            def fire(q):
                a, b, sem = cols[q % 2]
                src = pl.ds(q * PASS, PASS)
                cps[q % 2] = (
                    pltpu.async_copy(tbl_a.at[d, src], a, sem),
                    pltpu.async_copy(tbl_b.at[d, src], b, sem),
                )

            fire(0)
            for q in range(NPASS):
                cpa, cpb = cps[q % 2]
                cpa.wait()
                cpb.wait()
                if q + 1 < NPASS:
                    fire(q + 1)
                a, b, _ = cols[q % 2]
                base = q * PASS

                def body(v, carry):
                    o = v * L
                    iv = idx_v[pl.ds(o, L)]
                    li = jnp.clip(iv - base, 0, PASS - 1)
                    ga = plsc.load_gather(a, [li])
                    gb = plsc.load_gather(b, [li])
                    m = jnp.maximum(ga, gb)
                    prev = acc[pl.ds(o, L)]
                    keep = (iv >= base) & (iv < base + PASS)
                    acc[pl.ds(o, L)] = jnp.where(keep, m, prev)
                    return carry

                lax.fori_loop(0, VECS, body, 0)
            pltpu.sync_copy(acc, out.at[d])

    @pl.when(c == 0)
    def _():
        run_pair(user_h, suT, tuT, uT)

    @pl.when(c == 1)
    def _():
        run_pair(item_h, siT, tiT, iT)


@functools.lru_cache(maxsize=1)
def _sc_gather():
    return pl.kernel(
        _sc_gather_body,
        out_type=tuple(jax.ShapeDtypeStruct((EMB, BATCH), jnp.float32)
                       for _ in range(2)),
        mesh=plsc.VectorSubcoreMesh(core_axis_name="c", subcore_axis_name="s",
                                    num_cores=NC, num_subcores=NS),
        scratch_types=[
            pltpu.VMEM((BATCH,), jnp.int32),
            pltpu.VMEM((BATCH,), jnp.float32),
            pltpu.VMEM((PASS,), jnp.float32),
            pltpu.VMEM((PASS,), jnp.float32),
            pltpu.VMEM((PASS,), jnp.float32),
            pltpu.VMEM((PASS,), jnp.float32),
            pltpu.SemaphoreType.DMA,
            pltpu.SemaphoreType.DMA,
        ],
        compiler_params=pltpu.CompilerParams(use_tc_tiling_on_sc=False,
                                             needs_layout_passes=False),
    )


BLK = 2048


def _mlp_body(uT, iT, W1t, b1, W2t, b2, Wpt, bp, out):
    hT = jnp.concatenate((uT[...], iT[...]), axis=0)
    z = jnp.dot(W1t[...], hT, preferred_element_type=jnp.float32) + b1[...]
    z = jnp.maximum(z, 0.0)
    z = jnp.dot(W2t[...], z, preferred_element_type=jnp.float32) + b2[...]
    z = jnp.maximum(z, 0.0)
    o = jnp.dot(Wpt[...], z, preferred_element_type=jnp.float32) + bp[...]
    out[...] = jax.nn.sigmoid(o)


def _row_spec():
    return pl.BlockSpec((EMB, BLK), lambda i: (0, i))


def _full_spec(shape):
    return pl.BlockSpec(shape, lambda i: tuple(0 for _ in shape))


_mlp = pl.pallas_call(
    _mlp_body,
    grid=(BATCH // BLK,),
    in_specs=[
        _row_spec(), _row_spec(),
        _full_spec((128, 2 * EMB)), _full_spec((128, 1)),
        _full_spec((64, 128)), _full_spec((64, 1)),
        _full_spec((1, 64)), _full_spec((1, 1)),
    ],
    out_specs=pl.BlockSpec((1, BLK), lambda i: (0, i)),
    out_shape=jax.ShapeDtypeStruct((1, BATCH), jnp.float32),
)


@jax.jit
def kernel(x, su_emb, tu_emb, si_emb, ti_emb, W1, b1, W2, b2, Wp, bp):
    x = x.astype(jnp.int32)
    user = x[:, 0]
    item = x[:, 1]
    uT, iT = _sc_gather()(user, item, su_emb.T, tu_emb.T,
                          si_emb.T, ti_emb.T)
    out = _mlp(uT, iT,
               W1.T, b1.reshape(-1, 1), W2.T, b2.reshape(-1, 1),
               Wp.T, bp.reshape(1, 1))
    return out[0]


# speedup vs baseline: 5.2107x; 1.4973x over previous
import functools

import jax
import jax.numpy as jnp
from jax import lax
from jax.experimental import pallas as pl
from jax.experimental.pallas import tpu as pltpu
from jax.experimental.pallas import tpu_sc as plsc

VOCAB = 100000
EMB = 64
BATCH = 16384

NC = 2
NS = 16
NW = NC * NS
BPW = BATCH // NW
CH = 128
NCH = BPW // CH


def _sc_gather_body(idx_h, tbl_h, out, idx_v, rows, sem):
    c = lax.axis_index("c")
    s = lax.axis_index("s")
    wid = s * NC + c
    base = wid * BPW
    for j in range(NCH):
        pltpu.sync_copy(idx_h.at[pl.ds(base + j * CH, CH)], idx_v.at[j])
    cps = [pltpu.async_copy(tbl_h.at[idx_v.at[j]],
                            rows.at[pl.ds(j * CH, CH)], sem)
           for j in range(NCH)]
    for cp in cps:
        cp.wait()
    pltpu.sync_copy(rows, out.at[pl.ds(base, BPW)])


@functools.lru_cache(maxsize=1)
def _sc_gather():
    return pl.kernel(
        _sc_gather_body,
        out_type=jax.ShapeDtypeStruct((BATCH, 2 * EMB), jnp.float32),
        mesh=plsc.VectorSubcoreMesh(core_axis_name="c", subcore_axis_name="s",
                                    num_cores=NC, num_subcores=NS),
        scratch_types=[
            pltpu.VMEM((NCH, CH), jnp.int32),
            pltpu.VMEM((BPW, 2 * EMB), jnp.float32),
            pltpu.SemaphoreType.DMA,
        ],
        compiler_params=pltpu.CompilerParams(use_tc_tiling_on_sc=False),
    )


BLK = 2048


def _mlp_body(gu, gi, W1, b1, W2, b2, Wp, bp, out):
    ue = jnp.maximum(gu[:, :EMB], gu[:, EMB:])
    ie = jnp.maximum(gi[:, :EMB], gi[:, EMB:])
    h = jnp.concatenate((ue, ie), axis=1)
    h = jnp.dot(h, W1[...], preferred_element_type=jnp.float32) + b1[...]
    h = jnp.maximum(h, 0.0)
    h = jnp.dot(h, W2[...], preferred_element_type=jnp.float32) + b2[...]
    h = jnp.maximum(h, 0.0)
    o = jnp.dot(h, Wp[...], preferred_element_type=jnp.float32) + bp[...]
    out[...] = jax.nn.sigmoid(o)


def _row_spec():
    return pl.BlockSpec((BLK, 2 * EMB), lambda i: (i, 0))


def _full_spec(shape):
    return pl.BlockSpec(shape, lambda i: tuple(0 for _ in shape))


_mlp = pl.pallas_call(
    _mlp_body,
    grid=(BATCH // BLK,),
    in_specs=[
        _row_spec(), _row_spec(),
        _full_spec((2 * EMB, 128)), _full_spec((1, 128)),
        _full_spec((128, 64)), _full_spec((1, 64)),
        _full_spec((64, 1)), _full_spec((1, 1)),
    ],
    out_specs=pl.BlockSpec((BLK, 1), lambda i: (i, 0)),
    out_shape=jax.ShapeDtypeStruct((BATCH, 1), jnp.float32),
)


@jax.jit
def kernel(x, su_emb, tu_emb, si_emb, ti_emb, W1, b1, W2, b2, Wp, bp):
    x = x.astype(jnp.int32)
    user = x[:, 0]
    item = x[:, 1]
    ut = jnp.concatenate((su_emb, tu_emb), axis=1)
    gu = _sc_gather()(user, ut)
    it = jnp.concatenate((si_emb, ti_emb), axis=1)
    gi = _sc_gather()(item, it)
    out = _mlp(gu, gi,
               W1, b1.reshape(1, -1), W2, b2.reshape(1, -1),
               Wp, bp.reshape(1, 1))
    return out[:, 0]


# speedup vs baseline: 5.5769x; 1.0703x over previous
import functools

import jax
import jax.numpy as jnp
from jax import lax
from jax.experimental import pallas as pl
from jax.experimental.pallas import tpu as pltpu
from jax.experimental.pallas import tpu_sc as plsc

VOCAB = 100000
EMB = 64
BATCH = 16384

NC = 2
NS = 16
NW = NC * NS
BPW = BATCH // NW
CH = 128
NCH = BPW // CH


def _sc_gather_body(idx_h, tbl_h, out, idx_v, rows, sem):
    c = lax.axis_index("c")
    s = lax.axis_index("s")
    wid = s * NC + c
    base = wid * BPW
    for j in range(NCH):
        pltpu.sync_copy(idx_h.at[pl.ds(base + j * CH, CH)], idx_v.at[j])
    cps = [pltpu.async_copy(tbl_h.at[idx_v.at[j]],
                            rows.at[pl.ds(j * CH, CH)], sem)
           for j in range(NCH)]
    for cp in cps:
        cp.wait()
    pltpu.sync_copy(rows, out.at[pl.ds(base, BPW)])


@functools.lru_cache(maxsize=1)
def _sc_gather():
    return pl.kernel(
        _sc_gather_body,
        out_type=jax.ShapeDtypeStruct((BATCH, EMB), jnp.float32),
        mesh=plsc.VectorSubcoreMesh(core_axis_name="c", subcore_axis_name="s",
                                    num_cores=NC, num_subcores=NS),
        scratch_types=[
            pltpu.VMEM((NCH, CH), jnp.int32),
            pltpu.VMEM((BPW, EMB), jnp.float32),
            pltpu.SemaphoreType.DMA,
        ],
        compiler_params=pltpu.CompilerParams(use_tc_tiling_on_sc=False),
    )


BLK = 2048


def _mlp_body(gu, gi, W1, b1, W2, b2, Wp, bp, out):
    h = jnp.concatenate((gu[...], gi[...]), axis=1)
    h = jnp.dot(h, W1[...], preferred_element_type=jnp.float32) + b1[...]
    h = jnp.maximum(h, 0.0)
    h = jnp.dot(h, W2[...], preferred_element_type=jnp.float32) + b2[...]
    h = jnp.maximum(h, 0.0)
    o = jnp.dot(h, Wp[...], preferred_element_type=jnp.float32) + bp[...]
    out[...] = jax.nn.sigmoid(o)


def _row_spec():
    return pl.BlockSpec((BLK, EMB), lambda i: (i, 0))


def _full_spec(shape):
    return pl.BlockSpec(shape, lambda i: tuple(0 for _ in shape))


_mlp = pl.pallas_call(
    _mlp_body,
    grid=(BATCH // BLK,),
    in_specs=[
        _row_spec(), _row_spec(),
        _full_spec((2 * EMB, 128)), _full_spec((1, 128)),
        _full_spec((128, 64)), _full_spec((1, 64)),
        _full_spec((64, 1)), _full_spec((1, 1)),
    ],
    out_specs=pl.BlockSpec((BLK, 1), lambda i: (i, 0)),
    out_shape=jax.ShapeDtypeStruct((BATCH, 1), jnp.float32),
)


@jax.jit
def kernel(x, su_emb, tu_emb, si_emb, ti_emb, W1, b1, W2, b2, Wp, bp):
    x = x.astype(jnp.int32)
    user = x[:, 0]
    item = x[:, 1]
    mu = jnp.maximum(su_emb, tu_emb)
    gu = _sc_gather()(user, mu)
    mi = jnp.maximum(si_emb, ti_emb)
    gi = _sc_gather()(item, mi)
    out = _mlp(gu, gi,
               W1, b1.reshape(1, -1), W2, b2.reshape(1, -1),
               Wp, bp.reshape(1, 1))
    return out[:, 0]


# speedup vs baseline: 6.3532x; 1.1392x over previous
import functools

import jax
import jax.numpy as jnp
from jax import lax
from jax.experimental import pallas as pl
from jax.experimental.pallas import tpu as pltpu
from jax.experimental.pallas import tpu_sc as plsc

VOCAB = 100000
EMB = 64
BATCH = 16384

NC = 2
NS = 16
NW = NC * NS
BPW = BATCH // NW
CH = 128
NCH = BPW // CH


def _sc_gather_body(user_h, item_h, tbl_h, ou, oi, idx_u, idx_i, rows, sem):
    c = lax.axis_index("c")
    s = lax.axis_index("s")
    wid = s * NC + c
    base = wid * BPW
    for j in range(NCH):
        pltpu.sync_copy(user_h.at[pl.ds(base + j * CH, CH)], idx_u.at[j])
        pltpu.sync_copy(item_h.at[pl.ds(base + j * CH, CH)], idx_i.at[j])
    for idx, out in ((idx_u, ou), (idx_i, oi)):
        cps = [pltpu.async_copy(tbl_h.at[idx.at[j]],
                                rows.at[pl.ds(j * CH, CH)], sem)
               for j in range(NCH)]
        for cp in cps:
            cp.wait()
        pltpu.sync_copy(rows, out.at[pl.ds(base, BPW)])


@functools.lru_cache(maxsize=1)
def _sc_gather():
    return pl.kernel(
        _sc_gather_body,
        out_type=tuple(jax.ShapeDtypeStruct((BATCH, 2 * EMB), jnp.float32)
                       for _ in range(2)),
        mesh=plsc.VectorSubcoreMesh(core_axis_name="c", subcore_axis_name="s",
                                    num_cores=NC, num_subcores=NS),
        scratch_types=[
            pltpu.VMEM((NCH, CH), jnp.int32),
            pltpu.VMEM((NCH, CH), jnp.int32),
            pltpu.VMEM((BPW, 2 * EMB), jnp.float32),
            pltpu.SemaphoreType.DMA,
        ],
        compiler_params=pltpu.CompilerParams(use_tc_tiling_on_sc=False),
    )


BLK = 2048


def _mlp_body(gu, gi, W1, b1, W2, b2, Wp, bp, out):
    h = jnp.concatenate((gu[:, :EMB], gi[:, EMB:]), axis=1)
    h = jnp.dot(h, W1[...], preferred_element_type=jnp.float32) + b1[...]
    h = jnp.maximum(h, 0.0)
    h = jnp.dot(h, W2[...], preferred_element_type=jnp.float32) + b2[...]
    h = jnp.maximum(h, 0.0)
    o = jnp.dot(h, Wp[...], preferred_element_type=jnp.float32) + bp[...]
    out[...] = jax.nn.sigmoid(o)


def _row_spec():
    return pl.BlockSpec((BLK, 2 * EMB), lambda i: (i, 0))


def _full_spec(shape):
    return pl.BlockSpec(shape, lambda i: tuple(0 for _ in shape))


_mlp = pl.pallas_call(
    _mlp_body,
    grid=(BATCH // BLK,),
    in_specs=[
        _row_spec(), _row_spec(),
        _full_spec((2 * EMB, 128)), _full_spec((1, 128)),
        _full_spec((128, 64)), _full_spec((1, 64)),
        _full_spec((64, 1)), _full_spec((1, 1)),
    ],
    out_specs=pl.BlockSpec((BLK, 1), lambda i: (i, 0)),
    out_shape=jax.ShapeDtypeStruct((BATCH, 1), jnp.float32),
)


@jax.jit
def kernel(x, su_emb, tu_emb, si_emb, ti_emb, W1, b1, W2, b2, Wp, bp):
    x = x.astype(jnp.int32)
    user = x[:, 0]
    item = x[:, 1]
    M = jnp.concatenate((jnp.maximum(su_emb, tu_emb),
                         jnp.maximum(si_emb, ti_emb)), axis=1)
    gu, gi = _sc_gather()(user, item, M)
    out = _mlp(gu, gi,
               W1, b1.reshape(1, -1), W2, b2.reshape(1, -1),
               Wp, bp.reshape(1, 1))
    return out[:, 0]
